# Initial kernel scaffold; baseline (speedup 1.0000x reference)
import jax
import jax.numpy as jnp
from jax.experimental import pallas as pl


def kernel(x, edge_index, selections, W1, b1, g1, be1, W2, b2, g2, be2, W3, b3, g3, be3):
    raise NotImplementedError("write your pallas kernel here")


# speedup vs baseline: 11.6379x; 11.6379x over previous
import functools

import jax
import jax.numpy as jnp
from jax import lax
from jax.experimental import pallas as pl
from jax.experimental.pallas import tpu as pltpu
from jax.experimental.pallas import tpu_sc as plsc

N = 10000
E = 320000
D = 128
S = 9

NC = 2
NS = 16
L = 16
NW = NC * NS

EPW = E // NW
CHUNK = 80
NSTEPS = EPW // CHUNK
RPS = 624
REMR = N - NS * RPS
ZROWS = 156


def _sc_agg_body(table, srcs, sels, dsts, out,
                 src_v, sel_v, dst_v, gidx_v, rows_v, zbuf_v, acc_sh, sem):
    cid = lax.axis_index("c")
    sid = lax.axis_index("s")
    wid = cid * NS + sid

    def _zrow(i, carry):
        for j in range(D // L):
            zbuf_v[i, pl.ds(j * L, L)] = jnp.zeros((L,), jnp.float32)
        return carry
    lax.fori_loop(0, ZROWS, _zrow, 0)
    for k in range(RPS // ZROWS):
        pltpu.sync_copy(zbuf_v, acc_sh.at[pl.ds(sid * RPS + k * ZROWS, ZROWS)])

    @pl.when(sid == 0)
    def _zero_rem():
        pltpu.sync_copy(zbuf_v.at[pl.ds(0, REMR)],
                        acc_sh.at[pl.ds(NS * RPS, REMR)])
    plsc.subcore_barrier()

    def _step(i, carry):
        base = wid * EPW + i * CHUNK
        pltpu.sync_copy(srcs.at[pl.ds(base, CHUNK)], src_v)
        pltpu.sync_copy(sels.at[pl.ds(base, CHUNK)], sel_v)
        pltpu.sync_copy(dsts.at[pl.ds(base, CHUNK)], dst_v)
        for j in range(CHUNK // L):
            sl = pl.ds(j * L, L)
            gidx_v[sl] = src_v[sl] * S + sel_v[sl]
        pltpu.async_copy(table.at[gidx_v], rows_v, sem).wait()
        pltpu.sync_copy(rows_v, acc_sh.at[dst_v], add=True)
        return carry
    lax.fori_loop(0, NSTEPS, _step, 0)

    plsc.subcore_barrier()
    pltpu.sync_copy(acc_sh.at[pl.ds(sid * RPS, RPS)],
                    out.at[cid, pl.ds(sid * RPS, RPS)])

    @pl.when(sid == 0)
    def _out_rem():
        pltpu.sync_copy(acc_sh.at[pl.ds(NS * RPS, REMR)],
                        out.at[cid, pl.ds(NS * RPS, REMR)])


def _sc_agg(table, srcs, sels, dsts):
    mesh = plsc.VectorSubcoreMesh(core_axis_name="c", subcore_axis_name="s")
    f = functools.partial(
        pl.kernel,
        mesh=mesh,
        out_type=jax.ShapeDtypeStruct((NC, N, D), jnp.float32),
        scratch_types=[
            pltpu.VMEM((CHUNK,), jnp.int32),
            pltpu.VMEM((CHUNK,), jnp.int32),
            pltpu.VMEM((CHUNK,), jnp.int32),
            pltpu.VMEM((CHUNK,), jnp.int32),
            pltpu.VMEM((CHUNK, D), jnp.float32),
            pltpu.VMEM((ZROWS, D), jnp.float32),
            pltpu.VMEM_SHARED((N, D), jnp.float32),
            pltpu.SemaphoreType.DMA,
        ],
    )(_sc_agg_body)
    return f(table, srcs, sels, dsts)


BM = 400


def _mm_body(h_ref, w_ref, o_ref):
    o_ref[...] = jnp.dot(h_ref[...], w_ref[...],
                         preferred_element_type=jnp.float32)


def _matmul(h, Wr):
    K = Wr.shape[1]
    return pl.pallas_call(
        _mm_body,
        grid=(N // BM,),
        in_specs=[
            pl.BlockSpec((BM, D), lambda i: (i, 0)),
            pl.BlockSpec((D, K), lambda i: (0, 0)),
        ],
        out_specs=pl.BlockSpec((BM, K), lambda i: (i, 0)),
        out_shape=jax.ShapeDtypeStruct((N, K), jnp.float32),
    )(h, Wr)


def _elu(x):
    return jnp.where(x > 0, x, jnp.exp(jnp.minimum(x, 0.0)) - 1.0)


def _bn_elu_body(parts_ref, b_ref, g_ref, be_ref, o_ref, stat_ref):
    phase = pl.program_id(0)
    blk = pl.program_id(1)
    a = parts_ref[0] + parts_ref[1] + b_ref[...]

    @pl.when(jnp.logical_and(phase == 0, blk == 0))
    def _init():
        stat_ref[...] = jnp.zeros_like(stat_ref)

    @pl.when(phase == 0)
    def _acc():
        stat_ref[0:1, :] += jnp.sum(a, axis=0, keepdims=True)
        stat_ref[1:2, :] += jnp.sum(a * a, axis=0, keepdims=True)

    @pl.when(phase == 1)
    def _apply():
        m = stat_ref[0:1, :] / N
        v = stat_ref[1:2, :] / N - m * m
        xn = g_ref[...] * (a - m) * lax.rsqrt(v + 1e-5) + be_ref[...]
        o_ref[...] = _elu(xn)


def _bn_elu(parts, b, g, be):
    b2 = b.reshape(1, D)
    g2 = g.reshape(1, D)
    be2 = be.reshape(1, D)
    return pl.pallas_call(
        _bn_elu_body,
        grid=(2, N // BM),
        in_specs=[
            pl.BlockSpec((NC, BM, D), lambda p, i: (0, i, 0)),
            pl.BlockSpec((1, D), lambda p, i: (0, 0)),
            pl.BlockSpec((1, D), lambda p, i: (0, 0)),
            pl.BlockSpec((1, D), lambda p, i: (0, 0)),
        ],
        out_specs=pl.BlockSpec((BM, D), lambda p, i: (i, 0)),
        out_shape=jax.ShapeDtypeStruct((N, D), jnp.float32),
        scratch_shapes=[pltpu.VMEM((2, D), jnp.float32)],
    )(parts, b2, g2, be2)


def _final_body(parts_ref, b2_ref, g2_ref, be2_ref, x_ref, w3_ref, b3_ref,
                g3_ref, be3_ref, o_ref, s2_ref, s3_ref, y_ref):
    phase = pl.program_id(0)
    blk = pl.program_id(1)
    a = parts_ref[0] + parts_ref[1] + b2_ref[...]

    @pl.when(jnp.logical_and(phase == 0, blk == 0))
    def _init():
        s2_ref[...] = jnp.zeros_like(s2_ref)
        s3_ref[...] = jnp.zeros_like(s3_ref)

    @pl.when(phase == 0)
    def _acc2():
        s2_ref[0:1, :] += jnp.sum(a, axis=0, keepdims=True)
        s2_ref[1:2, :] += jnp.sum(a * a, axis=0, keepdims=True)

    @pl.when(phase == 1)
    def _mk_y():
        m = s2_ref[0:1, :] / N
        v = s2_ref[1:2, :] / N - m * m
        h2 = _elu(g2_ref[...] * (a - m) * lax.rsqrt(v + 1e-5) + be2_ref[...])
        y = h2 + jnp.dot(x_ref[...], w3_ref[...],
                         preferred_element_type=jnp.float32) + b3_ref[...]
        y_ref[pl.ds(blk * BM, BM), :] = y
        s3_ref[0:1, :] += jnp.sum(y, axis=0, keepdims=True)
        s3_ref[1:2, :] += jnp.sum(y * y, axis=0, keepdims=True)

    @pl.when(phase == 2)
    def _apply():
        m = s3_ref[0:1, :] / N
        v = s3_ref[1:2, :] / N - m * m
        y = y_ref[pl.ds(blk * BM, BM), :]
        o_ref[...] = _elu(g3_ref[...] * (y - m) * lax.rsqrt(v + 1e-5)
                          + be3_ref[...])


def _final(parts, b2, g2, be2, x, W3, b3, g3, be3):
    vecs = [v.reshape(1, D) for v in (b2, g2, be2, b3, g3, be3)]
    vspec = pl.BlockSpec((1, D), lambda p, i: (0, 0))
    return pl.pallas_call(
        _final_body,
        grid=(3, N // BM),
        in_specs=[
            pl.BlockSpec((NC, BM, D), lambda p, i: (0, i, 0)),
            vspec, vspec, vspec,
            pl.BlockSpec((BM, D), lambda p, i: (i, 0)),
            pl.BlockSpec((D, D), lambda p, i: (0, 0)),
            vspec, vspec, vspec,
        ],
        out_specs=pl.BlockSpec((BM, D), lambda p, i: (i, 0)),
        out_shape=jax.ShapeDtypeStruct((N, D), jnp.float32),
        scratch_shapes=[
            pltpu.VMEM((2, D), jnp.float32),
            pltpu.VMEM((2, D), jnp.float32),
            pltpu.VMEM((N, D), jnp.float32),
        ],
    )(parts, vecs[0], vecs[1], vecs[2], x, W3, vecs[3], vecs[4], vecs[5])


def kernel(x, edge_index, selections, W1, b1, g1, be1, W2, b2, g2, be2,
           W3, b3, g3, be3):
    src = edge_index[0].astype(jnp.int32)
    dst = edge_index[1].astype(jnp.int32)
    sel = selections.astype(jnp.int32)

    Wr1 = W1.transpose(1, 0, 2).reshape(D, S * D)
    Wr2 = W2.transpose(1, 0, 2).reshape(D, S * D)

    xw1 = _matmul(x, Wr1).reshape(N * S, D)
    parts1 = _sc_agg(xw1, src, sel, dst)
    h1 = _bn_elu(parts1, b1, g1, be1)

    xw2 = _matmul(h1, Wr2).reshape(N * S, D)
    parts2 = _sc_agg(xw2, src, sel, dst)
    return _final(parts2, b2, g2, be2, x, W3, b3, g3, be3)


# speedup vs baseline: 21.9156x; 1.8831x over previous
import functools

import jax
import jax.numpy as jnp
from jax import lax
from jax.experimental import pallas as pl
from jax.experimental.pallas import tpu as pltpu
from jax.experimental.pallas import tpu_sc as plsc

N = 10000
E = 320000
D = 128
S = 9

NC = 2
NS = 16
L = 16
NW = NC * NS

EPW = E // NW
CHUNK = 80
NSTEPS = EPW // CHUNK
RPS = 624
REMR = N - NS * RPS
ZROWS = 156


SELQ = 2000


def _sc_agg_body(table, srcs, sels, dsts, zrows, out,
                 gidx_v, selq_v, dst0_v, dst1_v, rows0_v, rows1_v, acc_sh,
                 sem0, sem1, semd0, semd1):
    cid = lax.axis_index("c")
    sid = lax.axis_index("s")
    wid = cid * NS + sid
    ebase = wid * EPW

    pltpu.sync_copy(zrows, acc_sh.at[pl.ds(sid * RPS, RPS)])

    @pl.when(sid == 0)
    def _zero_rem():
        pltpu.sync_copy(zrows.at[pl.ds(0, REMR)],
                        acc_sh.at[pl.ds(NS * RPS, REMR)])

    pltpu.sync_copy(srcs.at[pl.ds(ebase, EPW)], gidx_v)
    for q in range(EPW // SELQ):
        pltpu.sync_copy(sels.at[pl.ds(ebase + q * SELQ, SELQ)], selq_v)

        def _gix(k, carry):
            sl = pl.ds(q * SELQ + k * L, L)
            gidx_v[sl] = gidx_v[sl] * S + selq_v[pl.ds(k * L, L)]
            return carry
        lax.fori_loop(0, SELQ // L, _gix, 0)
    plsc.subcore_barrier()

    def _gather(c, buf, sem):
        return pltpu.make_async_copy(
            table.at[gidx_v.at[pl.ds(c * CHUNK, CHUNK)]], buf, sem)

    def _dstcp(c, buf, sem):
        return pltpu.make_async_copy(
            dsts.at[pl.ds(ebase + c * CHUNK, CHUNK)], buf, sem)

    def _scatter(buf, dbuf):
        pltpu.sync_copy(buf, acc_sh.at[dbuf], add=True)

    _gather(0, rows0_v, sem0).start()
    _dstcp(0, dst0_v, semd0).start()

    def _step(k, carry):
        c0 = 2 * k
        _gather(c0, rows0_v, sem0).wait()
        _dstcp(c0, dst0_v, semd0).wait()
        _gather(c0 + 1, rows1_v, sem1).start()
        _dstcp(c0 + 1, dst1_v, semd1).start()
        _scatter(rows0_v, dst0_v)
        _gather(c0 + 2, rows0_v, sem0).start()
        _dstcp(c0 + 2, dst0_v, semd0).start()
        _gather(c0 + 1, rows1_v, sem1).wait()
        _dstcp(c0 + 1, dst1_v, semd1).wait()
        _scatter(rows1_v, dst1_v)
        return carry
    lax.fori_loop(0, (NSTEPS - 1) // 2, _step, 0)

    _gather(NSTEPS - 1, rows0_v, sem0).wait()
    _dstcp(NSTEPS - 1, dst0_v, semd0).wait()
    _scatter(rows0_v, dst0_v)

    plsc.subcore_barrier()
    pltpu.sync_copy(acc_sh.at[pl.ds(sid * RPS, RPS)],
                    out.at[cid, pl.ds(sid * RPS, RPS)])

    @pl.when(sid == 0)
    def _out_rem():
        pltpu.sync_copy(acc_sh.at[pl.ds(NS * RPS, REMR)],
                        out.at[cid, pl.ds(NS * RPS, REMR)])


def _sc_agg(table, srcs, sels, dsts, zrows):
    mesh = plsc.VectorSubcoreMesh(core_axis_name="c", subcore_axis_name="s")
    f = functools.partial(
        pl.kernel,
        mesh=mesh,
        out_type=jax.ShapeDtypeStruct((NC, N, D), jnp.float32),
        scratch_types=[
            pltpu.VMEM((EPW,), jnp.int32),
            pltpu.VMEM((SELQ,), jnp.int32),
            pltpu.VMEM((CHUNK,), jnp.int32),
            pltpu.VMEM((CHUNK,), jnp.int32),
            pltpu.VMEM((CHUNK, D), jnp.float32),
            pltpu.VMEM((CHUNK, D), jnp.float32),
            pltpu.VMEM_SHARED((N, D), jnp.float32),
            pltpu.SemaphoreType.DMA,
            pltpu.SemaphoreType.DMA,
            pltpu.SemaphoreType.DMA,
            pltpu.SemaphoreType.DMA,
        ],
    )(_sc_agg_body)
    return f(table, srcs, sels, dsts, zrows)


BM = 400


def _mm_body(h_ref, w_ref, o_ref):
    o_ref[...] = jnp.dot(h_ref[...], w_ref[...],
                         preferred_element_type=jnp.float32)


def _matmul(h, Wr):
    K = Wr.shape[1]
    return pl.pallas_call(
        _mm_body,
        grid=(N // BM,),
        in_specs=[
            pl.BlockSpec((BM, D), lambda i: (i, 0)),
            pl.BlockSpec((D, K), lambda i: (0, 0)),
        ],
        out_specs=pl.BlockSpec((BM, K), lambda i: (i, 0)),
        out_shape=jax.ShapeDtypeStruct((N, K), jnp.float32),
    )(h, Wr)


def _elu(x):
    return jnp.where(x > 0, x, jnp.exp(jnp.minimum(x, 0.0)) - 1.0)


def _bn_elu_body(parts_ref, b_ref, g_ref, be_ref, o_ref, stat_ref):
    phase = pl.program_id(0)
    blk = pl.program_id(1)
    a = parts_ref[0] + parts_ref[1] + b_ref[...]

    @pl.when(jnp.logical_and(phase == 0, blk == 0))
    def _init():
        stat_ref[...] = jnp.zeros_like(stat_ref)

    @pl.when(phase == 0)
    def _acc():
        stat_ref[0:1, :] += jnp.sum(a, axis=0, keepdims=True)
        stat_ref[1:2, :] += jnp.sum(a * a, axis=0, keepdims=True)

    @pl.when(phase == 1)
    def _apply():
        m = stat_ref[0:1, :] / N
        v = stat_ref[1:2, :] / N - m * m
        xn = g_ref[...] * (a - m) * lax.rsqrt(v + 1e-5) + be_ref[...]
        o_ref[...] = _elu(xn)


def _bn_elu(parts, b, g, be):
    b2 = b.reshape(1, D)
    g2 = g.reshape(1, D)
    be2 = be.reshape(1, D)
    return pl.pallas_call(
        _bn_elu_body,
        grid=(2, N // BM),
        in_specs=[
            pl.BlockSpec((NC, BM, D), lambda p, i: (0, i, 0)),
            pl.BlockSpec((1, D), lambda p, i: (0, 0)),
            pl.BlockSpec((1, D), lambda p, i: (0, 0)),
            pl.BlockSpec((1, D), lambda p, i: (0, 0)),
        ],
        out_specs=pl.BlockSpec((BM, D), lambda p, i: (i, 0)),
        out_shape=jax.ShapeDtypeStruct((N, D), jnp.float32),
        scratch_shapes=[pltpu.VMEM((2, D), jnp.float32)],
    )(parts, b2, g2, be2)


def _final_body(parts_ref, b2_ref, g2_ref, be2_ref, x_ref, w3_ref, b3_ref,
                g3_ref, be3_ref, o_ref, s2_ref, s3_ref, y_ref):
    phase = pl.program_id(0)
    blk = pl.program_id(1)
    a = parts_ref[0] + parts_ref[1] + b2_ref[...]

    @pl.when(jnp.logical_and(phase == 0, blk == 0))
    def _init():
        s2_ref[...] = jnp.zeros_like(s2_ref)
        s3_ref[...] = jnp.zeros_like(s3_ref)

    @pl.when(phase == 0)
    def _acc2():
        s2_ref[0:1, :] += jnp.sum(a, axis=0, keepdims=True)
        s2_ref[1:2, :] += jnp.sum(a * a, axis=0, keepdims=True)

    @pl.when(phase == 1)
    def _mk_y():
        m = s2_ref[0:1, :] / N
        v = s2_ref[1:2, :] / N - m * m
        h2 = _elu(g2_ref[...] * (a - m) * lax.rsqrt(v + 1e-5) + be2_ref[...])
        y = h2 + jnp.dot(x_ref[...], w3_ref[...],
                         preferred_element_type=jnp.float32) + b3_ref[...]
        y_ref[pl.ds(blk * BM, BM), :] = y
        s3_ref[0:1, :] += jnp.sum(y, axis=0, keepdims=True)
        s3_ref[1:2, :] += jnp.sum(y * y, axis=0, keepdims=True)

    @pl.when(phase == 2)
    def _apply():
        m = s3_ref[0:1, :] / N
        v = s3_ref[1:2, :] / N - m * m
        y = y_ref[pl.ds(blk * BM, BM), :]
        o_ref[...] = _elu(g3_ref[...] * (y - m) * lax.rsqrt(v + 1e-5)
                          + be3_ref[...])


def _final(parts, b2, g2, be2, x, W3, b3, g3, be3):
    vecs = [v.reshape(1, D) for v in (b2, g2, be2, b3, g3, be3)]
    vspec = pl.BlockSpec((1, D), lambda p, i: (0, 0))
    return pl.pallas_call(
        _final_body,
        grid=(3, N // BM),
        in_specs=[
            pl.BlockSpec((NC, BM, D), lambda p, i: (0, i, 0)),
            vspec, vspec, vspec,
            pl.BlockSpec((BM, D), lambda p, i: (i, 0)),
            pl.BlockSpec((D, D), lambda p, i: (0, 0)),
            vspec, vspec, vspec,
        ],
        out_specs=pl.BlockSpec((BM, D), lambda p, i: (i, 0)),
        out_shape=jax.ShapeDtypeStruct((N, D), jnp.float32),
        scratch_shapes=[
            pltpu.VMEM((2, D), jnp.float32),
            pltpu.VMEM((2, D), jnp.float32),
            pltpu.VMEM((N, D), jnp.float32),
        ],
    )(parts, vecs[0], vecs[1], vecs[2], x, W3, vecs[3], vecs[4], vecs[5])


def kernel(x, edge_index, selections, W1, b1, g1, be1, W2, b2, g2, be2,
           W3, b3, g3, be3):
    src = edge_index[0].astype(jnp.int32)
    dst = edge_index[1].astype(jnp.int32)
    sel = selections.astype(jnp.int32)

    zrows = jnp.zeros((RPS, D), jnp.float32)

    Wr1 = W1.transpose(1, 0, 2).reshape(D, S * D)
    Wr2 = W2.transpose(1, 0, 2).reshape(D, S * D)

    xw1 = _matmul(x, Wr1).reshape(N * S, D)
    parts1 = _sc_agg(xw1, src, sel, dst, zrows)
    h1 = _bn_elu(parts1, b1, g1, be1)

    xw2 = _matmul(h1, Wr2).reshape(N * S, D)
    parts2 = _sc_agg(xw2, src, sel, dst, zrows)
    return _final(parts2, b2, g2, be2, x, W3, b3, g3, be3)
